# baseline (speedup 1.0000x reference)
import jax
import jax.numpy as jnp
from jax import lax
from jax.experimental import pallas as pl
from jax.experimental.pallas import tpu as pltpu
from jax.experimental.pallas import tpu_sc as plsc

N_NODES = 10000
SN = 10112
NFEAT = 128
NHID = 32
RELDIM = 64
NRELA = 500
RP = 512
ALPHA = 0.2

E1 = 160000
E1P = 163840
E1W = E1P // 32
E2 = 50000
E2P = 53248
E2W = E2P // 32
C = 128
ROWW = 72
NSTRIPES = SN // C

_f32 = jnp.float32
_i32 = jnp.int32


def _tc_a1(x0, rel, ah0, ah1, a2h0, a2h1, aout, a2out, W,
           usrc, udst, t3q, rfull, r2q, ortho):
    A1 = jnp.concatenate([ah0[:, :NFEAT], ah1[:, :NFEAT]], axis=0)
    A2 = jnp.concatenate([ah0[:, NFEAT:2 * NFEAT], ah1[:, NFEAT:2 * NFEAT]], axis=0)
    A3 = jnp.concatenate([ah0[:, 2 * NFEAT:], ah1[:, 2 * NFEAT:]], axis=0)

    dn = (((1,), (1,)), ((), ()))

    def table2(rows, out_ref, npad):
        s0 = lax.dot_general(rows[:, :NHID], a2h0[...], (((1,), (1,)), ((), ())))
        s1 = lax.dot_general(rows[:, NHID:], a2h1[...], (((1,), (1,)), ((), ())))
        n = rows.shape[0]
        pad = jnp.zeros((n, ROWW - 66), _f32)
        t = jnp.concatenate([rows, s0, s1, pad], axis=1)
        out_ref[...] = jnp.concatenate(
            [t, jnp.zeros((npad, ROWW), _f32)], axis=0)

    Us = lax.dot_general(x0[...], A1, dn)
    Ud = lax.dot_general(x0[...], A2, dn)
    table2(Us, usrc, SN - N_NODES)
    table2(Ud, udst, SN - N_NODES)

    T3 = lax.dot_general(rel[...], A3, dn)
    table2(T3, t3q, RP - NRELA)

    R = lax.dot_general(rel[...], W[...], (((1,), (0,)), ((), ())))
    rfull[...] = R
    B3 = aout[:, 2 * RELDIM:]
    R2 = lax.dot_general(R, B3, dn)
    sO = lax.dot_general(R2, a2out[...], (((1,), (1,)), ((), ())))
    r2q[...] = jnp.concatenate([
        jnp.concatenate([R2, sO, jnp.zeros((NRELA, ROWW - 65), _f32)], axis=1),
        jnp.zeros((RP - NRELA, ROWW), _f32)], axis=0)

    tot = jnp.float32(0.0)
    for a in (ah0[...], ah1[...], aout[...]):
        hd = a.shape[0] // 2
        ahh = a.reshape(2, hd, a.shape[1])
        gram = lax.dot_general(ahh, ahh, (((2,), (2,)), ((0,), (0,))))
        ii = lax.broadcasted_iota(_i32, (hd, hd), 0)
        jj = lax.broadcasted_iota(_i32, (hd, hd), 1)
        eye = jnp.where(ii == jj, jnp.float32(1.0), jnp.float32(0.0))
        tot = tot + 0.01 * jnp.sum((gram - eye[None]) ** 2)
    ortho[...] = jnp.reshape(tot, (1, 1))


def _tc_a2(eeb, a2h0, a2h1, ah0, ah1, gq):
    A3 = jnp.concatenate([ah0[:, 2 * NFEAT:], ah1[:, 2 * NFEAT:]], axis=0)
    dn = (((1,), (1,)), ((), ()))
    G = lax.dot_general(eeb[...], A3, dn)
    s0 = lax.dot_general(G[:, :NHID], a2h0[...], dn)
    s1 = lax.dot_general(G[:, NHID:], a2h1[...], dn)
    pad = jnp.zeros((G.shape[0], ROWW - 66), _f32)
    gq[...] = jnp.concatenate([G, s0, s1, pad], axis=1)


def _tc_b(acc, aout, a2out, vsrc, vdst):
    s = acc[0] + acc[1]
    w0 = s[:, 64:65]
    w1 = s[:, 65:66]
    w0 = jnp.where(w0 == 0.0, jnp.float32(1e-12), w0)
    w1 = jnp.where(w1 == 0.0, jnp.float32(1e-12), w1)
    h0 = s[:, :NHID] / w0
    h1 = s[:, NHID:2 * NHID] / w1
    x1 = jnp.concatenate([_elu(h0), _elu(h1)], axis=1)
    dn = (((1,), (1,)), ((), ()))
    B1 = aout[:, :RELDIM]
    B2 = aout[:, RELDIM:2 * RELDIM]
    pad = jnp.zeros((SN, ROWW - 65), _f32)
    Vs = lax.dot_general(x1, B1, dn)
    Vd = lax.dot_general(x1, B2, dn)
    vsrc[...] = jnp.concatenate(
        [Vs, lax.dot_general(Vs, a2out[...], dn), pad], axis=1)
    vdst[...] = jnp.concatenate(
        [Vd, lax.dot_general(Vd, a2out[...], dn), pad], axis=1)


def _elu(x):
    return jnp.where(x > 0, x, jnp.exp(jnp.minimum(x, 0.0)) - 1.0)


def _tc_c(acc, out):
    s = acc[0] + acc[1]
    w = s[:, 64:65]
    w = jnp.where(w == 0.0, jnp.float32(1e-12), w)
    out[...] = (s[:, :RELDIM] / w)[:N_NODES, :]


LINEAR, G1, G2 = 0, 1, 2


def _compute_chunk(mode, nheads, srcv, rowsA, rowsB, qa, qb, outr, wb, accsh):
    seg = 64 // nheads
    lanes = lax.broadcasted_iota(_i32, (16,), 0)
    for g in range(C // 16):
        e16 = g * 16 + lanes
        for h in range(nheads):
            c16 = jnp.full((16,), 64 + h, _i32)
            sc = plsc.load_gather(rowsA, [e16, c16]) \
                + plsc.load_gather(rowsB, [e16, c16]) \
                + plsc.load_gather(qa, [e16, c16])
            if mode == G2:
                sc = sc + plsc.load_gather(qb, [e16, c16])
            w = jnp.exp(jnp.where(sc > 0, -sc, (-ALPHA) * sc))
            wb[h][pl.ds(g * 16, 16)] = w
            plsc.store_scatter(outr, [e16, c16], w)

    def edge_body(i4, _):
        for u in range(4):
            i = i4 * 4 + u
            for h in range(nheads):
                wv = plsc.load_gather(wb[h], [jnp.full((16,), i, _i32)])
                for jj in range(seg // 16):
                    j = h * (seg // 16) + jj
                    m = rowsA[i, pl.ds(j * 16, 16)] + rowsB[i, pl.ds(j * 16, 16)] \
                        + qa[i, pl.ds(j * 16, 16)]
                    if mode == G2:
                        m = m + qb[i, pl.ds(j * 16, 16)]
                    outr[i, pl.ds(j * 16, 16)] = wv * m
        return 0

    lax.fori_loop(0, C // 4, edge_body, 0)
    pltpu.sync_copy(outr, accsh.at[srcv], add=True)


def _make_sc_kernel(nheads, qmode_a):
    na = 2 if qmode_a == LINEAR else 3

    def body(*refs):
        if qmode_a == LINEAR:
            (idxa, gq, idxb, tq, us, ud, acc) = refs[:7]
            scr = refs[7:]
        else:
            (idxa, idxb, tq, us, ud, acc) = refs[:6]
            scr = refs[6:]
            gq = None
        (sdv0, sdv1, bdv,
         rowsA0, rowsB0, rowsA1, rowsB1, qa0, qa1, qb, outr) = scr[:11]
        scr = scr[11:]
        wb = list(scr[:nheads]); scr = scr[nheads:]
        accsh, sem0, sem1 = scr

        cid = lax.axis_index("c")
        sid = lax.axis_index("s")
        wid = cid * 16 + sid

        z16 = jnp.zeros((16,), _f32)

        def zero_body(i, _):
            for j in range(4):
                outr[i, pl.ds(j * 16, 16)] = z16
            outr[i, pl.ds(ROWW - 16, 16)] = z16
            return 0

        lax.fori_loop(0, C, zero_body, 0)

        for k in range((NSTRIPES + 15) // 16):
            stripe = sid + k * 16

            @pl.when(stripe < NSTRIPES)
            def _():
                pltpu.sync_copy(outr, accsh.at[pl.ds(stripe * C, C)])

        plsc.subcore_barrier()

        bufs = [
            (sdv0, rowsA0, rowsB0, qa0, sem0),
            (sdv1, rowsA1, rowsB1, qa1, sem1),
        ]

        nch_a = E1W // C
        assert nch_a % 2 == 0
        glin = qmode_a == LINEAR

        def issue(k, b):
            sdv, rowsA, rowsB, qa, sem = bufs[b]
            row = wid * nch_a + k
            pltpu.sync_copy(idxa.at[row], sdv)
            pltpu.async_copy(us.at[sdv.at[0]], rowsA, sem)
            pltpu.async_copy(ud.at[sdv.at[1]], rowsB, sem)
            if glin:
                base = pl.multiple_of(wid * E1W + k * C, C)
                pltpu.async_copy(gq.at[pl.ds(base, C)], qa, sem)
            else:
                pltpu.async_copy(tq.at[sdv.at[2]], qa, sem)

        def consume(b):
            sdv, rowsA, rowsB, qa, sem = bufs[b]
            pltpu.make_async_copy(us.at[sdv.at[0]], rowsA, sem).wait()
            pltpu.make_async_copy(ud.at[sdv.at[1]], rowsB, sem).wait()
            pltpu.make_async_copy(us.at[sdv.at[0]], qa, sem).wait()
            _compute_chunk(qmode_a, nheads, sdv.at[0], rowsA, rowsB, qa, qb,
                           outr, wb, accsh)

        issue(0, 0)

        def pair_body(p, _):
            k = p * 2
            issue(k + 1, 1)
            consume(0)

            @pl.when(k + 2 < nch_a)
            def _():
                issue(k + 2, 0)

            consume(1)
            return 0

        lax.fori_loop(0, nch_a // 2, pair_body, 0)

        nch_b = E2W // C

        def chunk_b(k, _):
            row = wid * nch_b + k
            pltpu.sync_copy(idxb.at[row], bdv)
            pltpu.async_copy(us.at[bdv.at[0]], rowsA0, sem0)
            pltpu.async_copy(ud.at[bdv.at[1]], rowsB0, sem0)
            pltpu.async_copy(tq.at[bdv.at[2]], qa0, sem0)
            pltpu.async_copy(tq.at[bdv.at[3]], qb, sem0)
            pltpu.make_async_copy(us.at[bdv.at[0]], rowsA0, sem0).wait()
            pltpu.make_async_copy(ud.at[bdv.at[1]], rowsB0, sem0).wait()
            pltpu.make_async_copy(us.at[bdv.at[0]], qa0, sem0).wait()
            pltpu.make_async_copy(us.at[bdv.at[0]], qb, sem0).wait()
            _compute_chunk(G2, nheads, bdv.at[0], rowsA0, rowsB0, qa0, qb,
                           outr, wb, accsh)
            return 0

        lax.fori_loop(0, nch_b, chunk_b, 0)

        plsc.subcore_barrier()

        for k in range((NSTRIPES + 15) // 16):
            stripe = sid + k * 16

            @pl.when(stripe < NSTRIPES)
            def _():
                pltpu.sync_copy(accsh.at[pl.ds(stripe * C, C)],
                                acc.at[cid, pl.ds(stripe * C, C)])

    scratch = [
        pltpu.VMEM((na, C), _i32), pltpu.VMEM((na, C), _i32),
        pltpu.VMEM((4, C), _i32),
        pltpu.VMEM((C, ROWW), _f32), pltpu.VMEM((C, ROWW), _f32),
        pltpu.VMEM((C, ROWW), _f32), pltpu.VMEM((C, ROWW), _f32),
        pltpu.VMEM((C, ROWW), _f32), pltpu.VMEM((C, ROWW), _f32),
        pltpu.VMEM((C, ROWW), _f32),
        pltpu.VMEM((C, ROWW), _f32),
    ]
    scratch += [pltpu.VMEM((C,), _f32)] * nheads
    scratch += [pltpu.VMEM_SHARED((SN, ROWW), _f32),
                pltpu.SemaphoreType.DMA, pltpu.SemaphoreType.DMA]

    mesh = plsc.VectorSubcoreMesh(core_axis_name="c", subcore_axis_name="s",
                                  num_cores=2, num_subcores=16)
    return pl.kernel(
        body,
        out_type=jax.ShapeDtypeStruct((2, SN, ROWW), _f32),
        mesh=mesh,
        scratch_types=scratch,
        compiler_params=pltpu.CompilerParams(needs_layout_passes=False,
                                             use_tc_tiling_on_sc=False),
    )


def kernel(Corpus_, batch_inputs, entity_embeddings, relation_embed, edge_list,
           edge_type, edge_embed, edge_list_nhop, edge_type_nhop, a_head0,
           a2_head0, a_head1, a2_head1, a_out, a2_out, W):
    srcA = jnp.concatenate([edge_list[0], jnp.full((E1P - E1,), N_NODES, _i32)]).astype(_i32)
    dstA = jnp.concatenate([edge_list[1], jnp.zeros((E1P - E1,), _i32)]).astype(_i32)
    tyA = jnp.concatenate([edge_type, jnp.zeros((E1P - E1,), _i32)]).astype(_i32)
    srcB = jnp.concatenate([edge_list_nhop[0], jnp.full((E2P - E2,), N_NODES, _i32)]).astype(_i32)
    dstB = jnp.concatenate([edge_list_nhop[1], jnp.zeros((E2P - E2,), _i32)]).astype(_i32)
    t0B = jnp.concatenate([edge_type_nhop[:, 0], jnp.zeros((E2P - E2,), _i32)]).astype(_i32)
    t1B = jnp.concatenate([edge_type_nhop[:, 1], jnp.zeros((E2P - E2,), _i32)]).astype(_i32)
    idxa1 = jnp.stack([srcA.reshape(-1, C), dstA.reshape(-1, C)], axis=1)
    idxa2 = jnp.stack([srcA.reshape(-1, C), dstA.reshape(-1, C),
                       tyA.reshape(-1, C)], axis=1)
    idxb = jnp.stack([srcB.reshape(-1, C), dstB.reshape(-1, C),
                      t0B.reshape(-1, C), t1B.reshape(-1, C)], axis=1)

    usrc, udst, t3q, rfull, r2q, ortho = pl.pallas_call(
        _tc_a1,
        out_shape=[
            jax.ShapeDtypeStruct((SN, ROWW), _f32),
            jax.ShapeDtypeStruct((SN, ROWW), _f32),
            jax.ShapeDtypeStruct((RP, ROWW), _f32),
            jax.ShapeDtypeStruct((NRELA, 64), _f32),
            jax.ShapeDtypeStruct((RP, ROWW), _f32),
            jax.ShapeDtypeStruct((1, 1), _f32),
        ],
    )(entity_embeddings, relation_embed, a_head0, a_head1, a2_head0, a2_head1,
      a_out, a2_out, W)

    BLK = 1280
    nblk = E1P // BLK
    nin = E1 // BLK
    gq1 = pl.pallas_call(
        _tc_a2,
        grid=(nblk,),
        in_specs=[
            pl.BlockSpec((BLK, RELDIM), lambda i: (jnp.minimum(i, nin - 1), 0)),
            pl.BlockSpec((1, NHID), lambda i: (0, 0)),
            pl.BlockSpec((1, NHID), lambda i: (0, 0)),
            pl.BlockSpec((NHID, 2 * NFEAT + RELDIM), lambda i: (0, 0)),
            pl.BlockSpec((NHID, 2 * NFEAT + RELDIM), lambda i: (0, 0)),
        ],
        out_specs=pl.BlockSpec((BLK, ROWW), lambda i: (i, 0)),
        out_shape=jax.ShapeDtypeStruct((E1P, ROWW), _f32),
    )(edge_embed, a2_head0, a2_head1, a_head0, a_head1)

    sc1 = _make_sc_kernel(2, LINEAR)
    acc1 = sc1(idxa1, gq1, idxb, t3q, usrc, udst)

    vsrc, vdst = pl.pallas_call(
        _tc_b,
        out_shape=[
            jax.ShapeDtypeStruct((SN, ROWW), _f32),
            jax.ShapeDtypeStruct((SN, ROWW), _f32),
        ],
    )(acc1, a_out, a2_out)

    sc2 = _make_sc_kernel(1, G1)
    acc2 = sc2(idxa2, idxb, r2q, vsrc, vdst)

    x = pl.pallas_call(
        _tc_c,
        out_shape=jax.ShapeDtypeStruct((N_NODES, 64), _f32),
    )(acc2)

    return (x, rfull, ortho[0, 0])


# speedup vs baseline: 1.0202x; 1.0202x over previous
import jax
import jax.numpy as jnp
from jax import lax
from jax.experimental import pallas as pl
from jax.experimental.pallas import tpu as pltpu
from jax.experimental.pallas import tpu_sc as plsc

N_NODES = 10000
SN = 10112
NFEAT = 128
NHID = 32
RELDIM = 64
NRELA = 500
RP = 512
ALPHA = 0.2

E1 = 160000
E1P = 163840
E1W = E1P // 32
E2 = 50000
E2P = 53248
E2W = E2P // 32
C = 128
ROWW = 72
NSTRIPES = SN // C

_f32 = jnp.float32
_i32 = jnp.int32


def _tc_a1(x0, rel, ah0, ah1, a2h0, a2h1, aout, a2out, W,
           usf, uss, udst, t3q, rfull, r2q, ortho):
    A1 = jnp.concatenate([ah0[:, :NFEAT], ah1[:, :NFEAT]], axis=0)
    A2 = jnp.concatenate([ah0[:, NFEAT:2 * NFEAT], ah1[:, NFEAT:2 * NFEAT]], axis=0)
    A3 = jnp.concatenate([ah0[:, 2 * NFEAT:], ah1[:, 2 * NFEAT:]], axis=0)

    dn = (((1,), (1,)), ((), ()))

    def table2(rows, out_ref, npad):
        s0 = lax.dot_general(rows[:, :NHID], a2h0[...], (((1,), (1,)), ((), ())))
        s1 = lax.dot_general(rows[:, NHID:], a2h1[...], (((1,), (1,)), ((), ())))
        n = rows.shape[0]
        pad = jnp.zeros((n, ROWW - 66), _f32)
        t = jnp.concatenate([rows, s0, s1, pad], axis=1)
        out_ref[...] = jnp.concatenate(
            [t, jnp.zeros((npad, ROWW), _f32)], axis=0)

    Us = lax.dot_general(x0[...], A1, dn)
    Ud = lax.dot_general(x0[...], A2, dn)
    usf[...] = jnp.concatenate(
        [Us, jnp.zeros((SN - N_NODES, 64), _f32)], axis=0)
    s0 = lax.dot_general(Us[:, :NHID], a2h0[...], (((1,), (1,)), ((), ())))
    s1 = lax.dot_general(Us[:, NHID:], a2h1[...], (((1,), (1,)), ((), ())))
    uss[...] = jnp.concatenate([
        jnp.concatenate([s0, s1, jnp.zeros((N_NODES, 6), _f32)], axis=1),
        jnp.zeros((SN - N_NODES, 8), _f32)], axis=0)
    table2(Ud, udst, SN - N_NODES)

    T3 = lax.dot_general(rel[...], A3, dn)
    table2(T3, t3q, RP - NRELA)

    R = lax.dot_general(rel[...], W[...], (((1,), (0,)), ((), ())))
    rfull[...] = R
    B3 = aout[:, 2 * RELDIM:]
    R2 = lax.dot_general(R, B3, dn)
    sO = lax.dot_general(R2, a2out[...], (((1,), (1,)), ((), ())))
    r2q[...] = jnp.concatenate([
        jnp.concatenate([R2, sO, jnp.zeros((NRELA, ROWW - 65), _f32)], axis=1),
        jnp.zeros((RP - NRELA, ROWW), _f32)], axis=0)

    tot = jnp.float32(0.0)
    for a in (ah0[...], ah1[...], aout[...]):
        hd = a.shape[0] // 2
        ahh = a.reshape(2, hd, a.shape[1])
        gram = lax.dot_general(ahh, ahh, (((2,), (2,)), ((0,), (0,))))
        ii = lax.broadcasted_iota(_i32, (hd, hd), 0)
        jj = lax.broadcasted_iota(_i32, (hd, hd), 1)
        eye = jnp.where(ii == jj, jnp.float32(1.0), jnp.float32(0.0))
        tot = tot + 0.01 * jnp.sum((gram - eye[None]) ** 2)
    ortho[...] = jnp.reshape(tot, (1, 1))


def _tc_a2(eeb, a2h0, a2h1, ah0, ah1, gq):
    A3 = jnp.concatenate([ah0[:, 2 * NFEAT:], ah1[:, 2 * NFEAT:]], axis=0)
    dn = (((1,), (1,)), ((), ()))
    G = lax.dot_general(eeb[...], A3, dn)
    s0 = lax.dot_general(G[:, :NHID], a2h0[...], dn)
    s1 = lax.dot_general(G[:, NHID:], a2h1[...], dn)
    pad = jnp.zeros((G.shape[0], ROWW - 66), _f32)
    gq[...] = jnp.concatenate([G, s0, s1, pad], axis=1)


def _tc_b(acc, usf, aout, a2out, vsf, vss, vdst):
    s = acc[0] + acc[1]
    w0r = s[:, 64:65]
    w1r = s[:, 65:66]
    w0 = jnp.where(w0r == 0.0, jnp.float32(1e-12), w0r)
    w1 = jnp.where(w1r == 0.0, jnp.float32(1e-12), w1r)
    u = usf[...]
    h0 = s[:, :NHID] / w0 + jnp.where(w0r == 0.0, 0.0, u[:, :NHID])
    h1 = s[:, NHID:2 * NHID] / w1 + jnp.where(w1r == 0.0, 0.0, u[:, NHID:])
    x1 = jnp.concatenate([_elu(h0), _elu(h1)], axis=1)
    dn = (((1,), (1,)), ((), ()))
    B1 = aout[:, :RELDIM]
    B2 = aout[:, RELDIM:2 * RELDIM]
    Vs = lax.dot_general(x1, B1, dn)
    Vd = lax.dot_general(x1, B2, dn)
    vsf[...] = Vs
    vss[...] = jnp.concatenate(
        [lax.dot_general(Vs, a2out[...], dn), jnp.zeros((SN, 7), _f32)], axis=1)
    vdst[...] = jnp.concatenate(
        [Vd, lax.dot_general(Vd, a2out[...], dn),
         jnp.zeros((SN, ROWW - 65), _f32)], axis=1)


def _elu(x):
    return jnp.where(x > 0, x, jnp.exp(jnp.minimum(x, 0.0)) - 1.0)


def _tc_c(acc, vsf, out):
    s = acc[0] + acc[1]
    wr = s[:, 64:65]
    w = jnp.where(wr == 0.0, jnp.float32(1e-12), wr)
    v = jnp.where(wr == 0.0, 0.0, vsf[...])
    out[...] = (s[:, :RELDIM] / w + v)[:N_NODES, :]


LINEAR, G1, G2 = 0, 1, 2


def _compute_chunk(mode, nheads, srcv, rowsA, rowsB, qa, qb, outr, wb, accsh):
    seg = 64 // nheads
    lanes = lax.broadcasted_iota(_i32, (16,), 0)
    for g in range(C // 16):
        e16 = g * 16 + lanes
        for h in range(nheads):
            c16 = jnp.full((16,), 64 + h, _i32)
            sc = plsc.load_gather(rowsA, [e16, jnp.full((16,), h, _i32)]) \
                + plsc.load_gather(rowsB, [e16, c16]) \
                + plsc.load_gather(qa, [e16, c16])
            if mode == G2:
                sc = sc + plsc.load_gather(qb, [e16, c16])
            w = jnp.exp(jnp.where(sc > 0, -sc, (-ALPHA) * sc))
            wb[h][pl.ds(g * 16, 16)] = w
            plsc.store_scatter(outr, [e16, c16], w)

    def edge_body(i4, _):
        for u in range(4):
            i = i4 * 4 + u
            for h in range(nheads):
                wv = plsc.load_gather(wb[h], [jnp.full((16,), i, _i32)])
                for jj in range(seg // 16):
                    j = h * (seg // 16) + jj
                    m = rowsB[i, pl.ds(j * 16, 16)] + qa[i, pl.ds(j * 16, 16)]
                    if mode == G2:
                        m = m + qb[i, pl.ds(j * 16, 16)]
                    outr[i, pl.ds(j * 16, 16)] = wv * m
        return 0

    lax.fori_loop(0, C // 4, edge_body, 0)
    pltpu.sync_copy(outr, accsh.at[srcv], add=True)


def _make_sc_kernel(nheads, qmode_a):
    na = 2 if qmode_a == LINEAR else 3

    def body(*refs):
        if qmode_a == LINEAR:
            (idxa, gq, idxb, tq, us, ud, acc) = refs[:7]
            scr = refs[7:]
        else:
            (idxa, idxb, tq, us, ud, acc) = refs[:6]
            scr = refs[6:]
            gq = None
        (sdv0, sdv1, bdv,
         rowsA0, rowsB0, rowsA1, rowsB1, qa0, qa1, qb, outr) = scr[:11]
        scr = scr[11:]
        wb = list(scr[:nheads]); scr = scr[nheads:]
        accsh, sem0, sem1 = scr

        cid = lax.axis_index("c")
        sid = lax.axis_index("s")
        wid = cid * 16 + sid

        z16 = jnp.zeros((16,), _f32)

        def zero_body(i, _):
            for j in range(4):
                outr[i, pl.ds(j * 16, 16)] = z16
            outr[i, pl.ds(ROWW - 16, 16)] = z16
            return 0

        lax.fori_loop(0, C, zero_body, 0)

        for k in range((NSTRIPES + 15) // 16):
            stripe = sid + k * 16

            @pl.when(stripe < NSTRIPES)
            def _():
                pltpu.sync_copy(outr, accsh.at[pl.ds(stripe * C, C)])

        plsc.subcore_barrier()

        bufs = [
            (sdv0, rowsA0, rowsB0, qa0, sem0),
            (sdv1, rowsA1, rowsB1, qa1, sem1),
        ]

        nch_a = E1W // C
        assert nch_a % 2 == 0
        glin = qmode_a == LINEAR

        def issue(k, b):
            sdv, rowsA, rowsB, qa, sem = bufs[b]
            row = wid * nch_a + k
            pltpu.sync_copy(idxa.at[row], sdv)
            pltpu.async_copy(us.at[sdv.at[0]], rowsA, sem)
            pltpu.async_copy(ud.at[sdv.at[1]], rowsB, sem)
            if glin:
                base = pl.multiple_of(wid * E1W + k * C, C)
                pltpu.async_copy(gq.at[pl.ds(base, C)], qa, sem)
            else:
                pltpu.async_copy(tq.at[sdv.at[2]], qa, sem)

        def consume(b):
            sdv, rowsA, rowsB, qa, sem = bufs[b]
            pltpu.make_async_copy(us.at[sdv.at[0]], rowsA, sem).wait()
            pltpu.make_async_copy(ud.at[sdv.at[1]], rowsB, sem).wait()
            pltpu.make_async_copy(us.at[sdv.at[0]], qa, sem).wait()
            _compute_chunk(qmode_a, nheads, sdv.at[0], rowsA, rowsB, qa, qb,
                           outr, wb, accsh)

        issue(0, 0)

        def pair_body(p, _):
            k = p * 2
            issue(k + 1, 1)
            consume(0)

            @pl.when(k + 2 < nch_a)
            def _():
                issue(k + 2, 0)

            consume(1)
            return 0

        lax.fori_loop(0, nch_a // 2, pair_body, 0)

        nch_b = E2W // C

        def chunk_b(k, _):
            row = wid * nch_b + k
            pltpu.sync_copy(idxb.at[row], bdv)
            pltpu.async_copy(us.at[bdv.at[0]], rowsA0, sem0)
            pltpu.async_copy(ud.at[bdv.at[1]], rowsB0, sem0)
            pltpu.async_copy(tq.at[bdv.at[2]], qa0, sem0)
            pltpu.async_copy(tq.at[bdv.at[3]], qb, sem0)
            pltpu.make_async_copy(us.at[bdv.at[0]], rowsA0, sem0).wait()
            pltpu.make_async_copy(ud.at[bdv.at[1]], rowsB0, sem0).wait()
            pltpu.make_async_copy(us.at[bdv.at[0]], qa0, sem0).wait()
            pltpu.make_async_copy(us.at[bdv.at[0]], qb, sem0).wait()
            _compute_chunk(G2, nheads, bdv.at[0], rowsA0, rowsB0, qa0, qb,
                           outr, wb, accsh)
            return 0

        lax.fori_loop(0, nch_b, chunk_b, 0)

        plsc.subcore_barrier()

        for k in range((NSTRIPES + 15) // 16):
            stripe = sid + k * 16

            @pl.when(stripe < NSTRIPES)
            def _():
                pltpu.sync_copy(accsh.at[pl.ds(stripe * C, C)],
                                acc.at[cid, pl.ds(stripe * C, C)])

    scratch = [
        pltpu.VMEM((na, C), _i32), pltpu.VMEM((na, C), _i32),
        pltpu.VMEM((4, C), _i32),
        pltpu.VMEM((C, 8), _f32), pltpu.VMEM((C, ROWW), _f32),
        pltpu.VMEM((C, 8), _f32), pltpu.VMEM((C, ROWW), _f32),
        pltpu.VMEM((C, ROWW), _f32), pltpu.VMEM((C, ROWW), _f32),
        pltpu.VMEM((C, ROWW), _f32),
        pltpu.VMEM((C, ROWW), _f32),
    ]
    scratch += [pltpu.VMEM((C,), _f32)] * nheads
    scratch += [pltpu.VMEM_SHARED((SN, ROWW), _f32),
                pltpu.SemaphoreType.DMA, pltpu.SemaphoreType.DMA]

    mesh = plsc.VectorSubcoreMesh(core_axis_name="c", subcore_axis_name="s",
                                  num_cores=2, num_subcores=16)
    return pl.kernel(
        body,
        out_type=jax.ShapeDtypeStruct((2, SN, ROWW), _f32),
        mesh=mesh,
        scratch_types=scratch,
        compiler_params=pltpu.CompilerParams(needs_layout_passes=False,
                                             use_tc_tiling_on_sc=False),
    )


def kernel(Corpus_, batch_inputs, entity_embeddings, relation_embed, edge_list,
           edge_type, edge_embed, edge_list_nhop, edge_type_nhop, a_head0,
           a2_head0, a_head1, a2_head1, a_out, a2_out, W):
    srcA = jnp.concatenate([edge_list[0], jnp.full((E1P - E1,), N_NODES, _i32)]).astype(_i32)
    dstA = jnp.concatenate([edge_list[1], jnp.zeros((E1P - E1,), _i32)]).astype(_i32)
    tyA = jnp.concatenate([edge_type, jnp.zeros((E1P - E1,), _i32)]).astype(_i32)
    srcB = jnp.concatenate([edge_list_nhop[0], jnp.full((E2P - E2,), N_NODES, _i32)]).astype(_i32)
    dstB = jnp.concatenate([edge_list_nhop[1], jnp.zeros((E2P - E2,), _i32)]).astype(_i32)
    t0B = jnp.concatenate([edge_type_nhop[:, 0], jnp.zeros((E2P - E2,), _i32)]).astype(_i32)
    t1B = jnp.concatenate([edge_type_nhop[:, 1], jnp.zeros((E2P - E2,), _i32)]).astype(_i32)
    idxa1 = jnp.stack([srcA.reshape(-1, C), dstA.reshape(-1, C)], axis=1)
    idxa2 = jnp.stack([srcA.reshape(-1, C), dstA.reshape(-1, C),
                       tyA.reshape(-1, C)], axis=1)
    idxb = jnp.stack([srcB.reshape(-1, C), dstB.reshape(-1, C),
                      t0B.reshape(-1, C), t1B.reshape(-1, C)], axis=1)

    usf, uss, udst, t3q, rfull, r2q, ortho = pl.pallas_call(
        _tc_a1,
        out_shape=[
            jax.ShapeDtypeStruct((SN, 64), _f32),
            jax.ShapeDtypeStruct((SN, 8), _f32),
            jax.ShapeDtypeStruct((SN, ROWW), _f32),
            jax.ShapeDtypeStruct((RP, ROWW), _f32),
            jax.ShapeDtypeStruct((NRELA, 64), _f32),
            jax.ShapeDtypeStruct((RP, ROWW), _f32),
            jax.ShapeDtypeStruct((1, 1), _f32),
        ],
    )(entity_embeddings, relation_embed, a_head0, a_head1, a2_head0, a2_head1,
      a_out, a2_out, W)

    BLK = 1280
    nblk = E1P // BLK
    nin = E1 // BLK
    gq1 = pl.pallas_call(
        _tc_a2,
        grid=(nblk,),
        in_specs=[
            pl.BlockSpec((BLK, RELDIM), lambda i: (jnp.minimum(i, nin - 1), 0)),
            pl.BlockSpec((1, NHID), lambda i: (0, 0)),
            pl.BlockSpec((1, NHID), lambda i: (0, 0)),
            pl.BlockSpec((NHID, 2 * NFEAT + RELDIM), lambda i: (0, 0)),
            pl.BlockSpec((NHID, 2 * NFEAT + RELDIM), lambda i: (0, 0)),
        ],
        out_specs=pl.BlockSpec((BLK, ROWW), lambda i: (i, 0)),
        out_shape=jax.ShapeDtypeStruct((E1P, ROWW), _f32),
    )(edge_embed, a2_head0, a2_head1, a_head0, a_head1)

    sc1 = _make_sc_kernel(2, LINEAR)
    acc1 = sc1(idxa1, gq1, idxb, t3q, uss, udst)

    vsf, vss, vdst = pl.pallas_call(
        _tc_b,
        out_shape=[
            jax.ShapeDtypeStruct((SN, 64), _f32),
            jax.ShapeDtypeStruct((SN, 8), _f32),
            jax.ShapeDtypeStruct((SN, ROWW), _f32),
        ],
    )(acc1, usf, a_out, a2_out)

    sc2 = _make_sc_kernel(1, G1)
    acc2 = sc2(idxa2, idxb, r2q, vss, vdst)

    x = pl.pallas_call(
        _tc_c,
        out_shape=jax.ShapeDtypeStruct((N_NODES, 64), _f32),
    )(acc2, vsf)

    return (x, rfull, ortho[0, 0])


# speedup vs baseline: 1.0846x; 1.0631x over previous
import jax
import jax.numpy as jnp
from jax import lax
from jax.experimental import pallas as pl
from jax.experimental.pallas import tpu as pltpu
from jax.experimental.pallas import tpu_sc as plsc

N_NODES = 10000
SN = 10112
NFEAT = 128
NHID = 32
RELDIM = 64
NRELA = 500
RP = 512
ALPHA = 0.2

E1 = 160000
E1P = 163840
E1W = E1P // 32
E2 = 50000
E2P = 53248
E2W = E2P // 32
C = 128
ROWW = 72
NSTRIPES = SN // C

_f32 = jnp.float32
_i32 = jnp.int32


def _tc_a1(x0, rel, ah0, ah1, a2h0, a2h1, aout, a2out, W,
           usf, uss, udst, t3q, rfull, r2q, ortho):
    A1 = jnp.concatenate([ah0[:, :NFEAT], ah1[:, :NFEAT]], axis=0)
    A2 = jnp.concatenate([ah0[:, NFEAT:2 * NFEAT], ah1[:, NFEAT:2 * NFEAT]], axis=0)
    A3 = jnp.concatenate([ah0[:, 2 * NFEAT:], ah1[:, 2 * NFEAT:]], axis=0)

    dn = (((1,), (1,)), ((), ()))

    def table2(rows, out_ref, npad):
        s0 = lax.dot_general(rows[:, :NHID], a2h0[...], (((1,), (1,)), ((), ())))
        s1 = lax.dot_general(rows[:, NHID:], a2h1[...], (((1,), (1,)), ((), ())))
        n = rows.shape[0]
        pad = jnp.zeros((n, ROWW - 66), _f32)
        t = jnp.concatenate([rows, s0, s1, pad], axis=1)
        out_ref[...] = jnp.concatenate(
            [t, jnp.zeros((npad, ROWW), _f32)], axis=0)

    Us = lax.dot_general(x0[...], A1, dn)
    Ud = lax.dot_general(x0[...], A2, dn)
    usf[...] = jnp.concatenate(
        [Us, jnp.zeros((SN - N_NODES, 64), _f32)], axis=0)
    s0 = lax.dot_general(Us[:, :NHID], a2h0[...], (((1,), (1,)), ((), ())))
    s1 = lax.dot_general(Us[:, NHID:], a2h1[...], (((1,), (1,)), ((), ())))
    uss[...] = jnp.concatenate([
        jnp.concatenate([s0, s1, jnp.zeros((N_NODES, 6), _f32)], axis=1),
        jnp.zeros((SN - N_NODES, 8), _f32)], axis=0)
    table2(Ud, udst, SN - N_NODES)

    T3 = lax.dot_general(rel[...], A3, dn)
    table2(T3, t3q, RP - NRELA)

    R = lax.dot_general(rel[...], W[...], (((1,), (0,)), ((), ())))
    rfull[...] = R
    B3 = aout[:, 2 * RELDIM:]
    R2 = lax.dot_general(R, B3, dn)
    sO = lax.dot_general(R2, a2out[...], (((1,), (1,)), ((), ())))
    r2q[...] = jnp.concatenate([
        jnp.concatenate([R2, sO, jnp.zeros((NRELA, ROWW - 65), _f32)], axis=1),
        jnp.zeros((RP - NRELA, ROWW), _f32)], axis=0)

    tot = jnp.float32(0.0)
    for a in (ah0[...], ah1[...], aout[...]):
        hd = a.shape[0] // 2
        ahh = a.reshape(2, hd, a.shape[1])
        gram = lax.dot_general(ahh, ahh, (((2,), (2,)), ((0,), (0,))))
        ii = lax.broadcasted_iota(_i32, (hd, hd), 0)
        jj = lax.broadcasted_iota(_i32, (hd, hd), 1)
        eye = jnp.where(ii == jj, jnp.float32(1.0), jnp.float32(0.0))
        tot = tot + 0.01 * jnp.sum((gram - eye[None]) ** 2)
    ortho[...] = jnp.reshape(tot, (1, 1))


def _tc_a2(eeb, a2h0, a2h1, ah0, ah1, gq):
    A3 = jnp.concatenate([ah0[:, 2 * NFEAT:], ah1[:, 2 * NFEAT:]], axis=0)
    dn = (((1,), (1,)), ((), ()))
    G = lax.dot_general(eeb[...], A3, dn)
    s0 = lax.dot_general(G[:, :NHID], a2h0[...], dn)
    s1 = lax.dot_general(G[:, NHID:], a2h1[...], dn)
    pad = jnp.zeros((G.shape[0], ROWW - 66), _f32)
    gq[...] = jnp.concatenate([G, s0, s1, pad], axis=1)


def _tc_b(acc, usf, aout, a2out, vsf, vss, vdst):
    s = acc[0] + acc[1]
    w0r = s[:, 64:65]
    w1r = s[:, 65:66]
    w0 = jnp.where(w0r == 0.0, jnp.float32(1e-12), w0r)
    w1 = jnp.where(w1r == 0.0, jnp.float32(1e-12), w1r)
    u = usf[...]
    h0 = s[:, :NHID] / w0 + jnp.where(w0r == 0.0, 0.0, u[:, :NHID])
    h1 = s[:, NHID:2 * NHID] / w1 + jnp.where(w1r == 0.0, 0.0, u[:, NHID:])
    x1 = jnp.concatenate([_elu(h0), _elu(h1)], axis=1)
    dn = (((1,), (1,)), ((), ()))
    B1 = aout[:, :RELDIM]
    B2 = aout[:, RELDIM:2 * RELDIM]
    Vs = lax.dot_general(x1, B1, dn)
    Vd = lax.dot_general(x1, B2, dn)
    vsf[...] = Vs
    vss[...] = jnp.concatenate(
        [lax.dot_general(Vs, a2out[...], dn), jnp.zeros((SN, 7), _f32)], axis=1)
    vdst[...] = jnp.concatenate(
        [Vd, lax.dot_general(Vd, a2out[...], dn),
         jnp.zeros((SN, ROWW - 65), _f32)], axis=1)


def _elu(x):
    return jnp.where(x > 0, x, jnp.exp(jnp.minimum(x, 0.0)) - 1.0)


def _tc_c(acc, vsf, out):
    s = acc[0] + acc[1]
    wr = s[:, 64:65]
    w = jnp.where(wr == 0.0, jnp.float32(1e-12), wr)
    v = jnp.where(wr == 0.0, 0.0, vsf[...])
    out[...] = (s[:, :RELDIM] / w + v)[:N_NODES, :]


LINEAR, G1, G2 = 0, 1, 2


def _compute_chunk(mode, nheads, srcv, rowsA, rowsB, qa, qb, outr, wb, accsh):
    seg = 64 // nheads
    lanes = lax.broadcasted_iota(_i32, (16,), 0)
    for g in range(C // 16):
        e16 = g * 16 + lanes
        for h in range(nheads):
            c16 = jnp.full((16,), 64 + h, _i32)
            sc = plsc.load_gather(rowsA, [e16, jnp.full((16,), h, _i32)]) \
                + plsc.load_gather(rowsB, [e16, c16]) \
                + plsc.load_gather(qa, [e16, c16])
            if mode == G2:
                sc = sc + plsc.load_gather(qb, [e16, c16])
            w = jnp.exp(jnp.where(sc > 0, -sc, (-ALPHA) * sc))
            wb[h][pl.ds(g * 16, 16)] = w
            plsc.store_scatter(outr, [e16, c16], w)

    def edge_body(i4, _):
        for u in range(4):
            i = i4 * 4 + u
            for h in range(nheads):
                wv = plsc.load_gather(wb[h], [jnp.full((16,), i, _i32)])
                for jj in range(seg // 16):
                    j = h * (seg // 16) + jj
                    m = rowsB[i, pl.ds(j * 16, 16)] + qa[i, pl.ds(j * 16, 16)]
                    if mode == G2:
                        m = m + qb[i, pl.ds(j * 16, 16)]
                    outr[i, pl.ds(j * 16, 16)] = wv * m
        return 0

    lax.fori_loop(0, C // 4, edge_body, 0)


def _make_sc_kernel(nheads, qmode_a):
    na = 2 if qmode_a == LINEAR else 3

    def body(*refs):
        if qmode_a == LINEAR:
            (idxa, gq, idxb, tq, us, ud, acc) = refs[:7]
            scr = refs[7:]
        else:
            (idxa, idxb, tq, us, ud, acc) = refs[:6]
            scr = refs[6:]
            gq = None
        (sdv0, sdv1, bdv0, bdv1, scidx0, scidx1,
         rowsA0, rowsB0, rowsA1, rowsB1, qa0, qa1, qb0, qb1,
         outr0, outr1) = scr[:16]
        scr = scr[16:]
        wb = list(scr[:nheads]); scr = scr[nheads:]
        accsh, sem0, sem1, semS0, semS1 = scr

        cid = lax.axis_index("c")
        sid = lax.axis_index("s")
        wid = cid * 16 + sid

        z16 = jnp.zeros((16,), _f32)

        def zero_body(i, _):
            for outr in (outr0, outr1):
                for j in range(4):
                    outr[i, pl.ds(j * 16, 16)] = z16
                outr[i, pl.ds(ROWW - 16, 16)] = z16
            return 0

        lax.fori_loop(0, C, zero_body, 0)

        for k in range((NSTRIPES + 15) // 16):
            stripe = sid + k * 16

            @pl.when(stripe < NSTRIPES)
            def _():
                pltpu.sync_copy(outr0, accsh.at[pl.ds(stripe * C, C)])

        plsc.subcore_barrier()

        bufsA = [
            (sdv0, rowsA0, rowsB0, qa0, qb0, sem0, outr0, semS0, scidx0),
            (sdv1, rowsA1, rowsB1, qa1, qb1, sem1, outr1, semS1, scidx1),
        ]
        bufsB = [
            (bdv0, rowsA0, rowsB0, qa0, qb0, sem0, outr0, semS0, scidx0),
            (bdv1, rowsA1, rowsB1, qa1, qb1, sem1, outr1, semS1, scidx1),
        ]

        nch_a = E1W // C
        nch_b = E2W // C
        assert nch_a % 2 == 0 and nch_b % 2 == 1
        glin = qmode_a == LINEAR

        def issue_a(k, b):
            sdv, rowsA, rowsB, qa, qb, sem, outr, semS, scidx = bufsA[b]
            row = wid * nch_a + k
            pltpu.sync_copy(idxa.at[row], sdv)
            pltpu.async_copy(us.at[sdv.at[0]], rowsA, sem)
            pltpu.async_copy(ud.at[sdv.at[1]], rowsB, sem)
            if glin:
                base = pl.multiple_of(wid * E1W + k * C, C)
                pltpu.async_copy(gq.at[pl.ds(base, C)], qa, sem)
            else:
                pltpu.async_copy(tq.at[sdv.at[2]], qa, sem)

        def issue_b(k, b):
            bdv, rowsA, rowsB, qa, qb, sem, outr, semS, scidx = bufsB[b]
            row = wid * nch_b + k
            pltpu.sync_copy(idxb.at[row], bdv)
            pltpu.async_copy(us.at[bdv.at[0]], rowsA, sem)
            pltpu.async_copy(ud.at[bdv.at[1]], rowsB, sem)
            pltpu.async_copy(tq.at[bdv.at[2]], qa, sem)
            pltpu.async_copy(tq.at[bdv.at[3]], qb, sem)

        def wait_scat(b, phase_bufs):
            idxv, rowsA, rowsB, qa, qb, sem, outr, semS, scidx = phase_bufs[b]
            pltpu.make_async_copy(outr, accsh.at[scidx], semS).wait()

        def consume(mode, b, phase_bufs):
            idxv, rowsA, rowsB, qa, qb, sem, outr, semS, scidx = phase_bufs[b]
            pltpu.make_async_copy(us.at[idxv.at[0]], rowsA, sem).wait()
            pltpu.make_async_copy(ud.at[idxv.at[1]], rowsB, sem).wait()
            pltpu.make_async_copy(us.at[idxv.at[0]], qa, sem).wait()
            if mode == G2:
                pltpu.make_async_copy(us.at[idxv.at[0]], qb, sem).wait()
            _compute_chunk(mode, nheads, idxv.at[0], rowsA, rowsB, qa, qb,
                           outr, wb, accsh)
            for j in range(C // 16):
                scidx[pl.ds(j * 16, 16)] = idxv[0, pl.ds(j * 16, 16)]
            pltpu.async_copy(outr, accsh.at[scidx], semS, add=True)

        issue_a(0, 0)

        def pair_a(p, _):
            k = p * 2
            issue_a(k + 1, 1)

            @pl.when(p > 0)
            def _():
                wait_scat(0, bufsA)

            consume(qmode_a, 0, bufsA)

            @pl.when(k + 2 < nch_a)
            def _():
                issue_a(k + 2, 0)

            @pl.when(p > 0)
            def _():
                wait_scat(1, bufsA)

            consume(qmode_a, 1, bufsA)
            return 0

        lax.fori_loop(0, nch_a // 2, pair_a, 0)

        issue_b(0, 0)

        def pair_b(q, _):
            k = q * 2
            issue_b(k + 1, 1)
            wait_scat(0, bufsB)
            consume(G2, 0, bufsB)

            @pl.when(k + 2 < nch_b)
            def _():
                issue_b(k + 2, 0)

            wait_scat(1, bufsB)
            consume(G2, 1, bufsB)
            return 0

        lax.fori_loop(0, nch_b // 2, pair_b, 0)
        wait_scat(0, bufsB)
        consume(G2, 0, bufsB)

        wait_scat(0, bufsB)
        wait_scat(1, bufsB)

        plsc.subcore_barrier()

        for k in range((NSTRIPES + 15) // 16):
            stripe = sid + k * 16

            @pl.when(stripe < NSTRIPES)
            def _():
                pltpu.sync_copy(accsh.at[pl.ds(stripe * C, C)],
                                acc.at[cid, pl.ds(stripe * C, C)])

    scratch = [
        pltpu.VMEM((na, C), _i32), pltpu.VMEM((na, C), _i32),
        pltpu.VMEM((4, C), _i32), pltpu.VMEM((4, C), _i32),
        pltpu.VMEM((C,), _i32), pltpu.VMEM((C,), _i32),
        pltpu.VMEM((C, 8), _f32), pltpu.VMEM((C, ROWW), _f32),
        pltpu.VMEM((C, 8), _f32), pltpu.VMEM((C, ROWW), _f32),
        pltpu.VMEM((C, ROWW), _f32), pltpu.VMEM((C, ROWW), _f32),
        pltpu.VMEM((C, ROWW), _f32), pltpu.VMEM((C, ROWW), _f32),
        pltpu.VMEM((C, ROWW), _f32), pltpu.VMEM((C, ROWW), _f32),
    ]
    scratch += [pltpu.VMEM((C,), _f32)] * nheads
    scratch += [pltpu.VMEM_SHARED((SN, ROWW), _f32),
                pltpu.SemaphoreType.DMA, pltpu.SemaphoreType.DMA,
                pltpu.SemaphoreType.DMA, pltpu.SemaphoreType.DMA]

    mesh = plsc.VectorSubcoreMesh(core_axis_name="c", subcore_axis_name="s",
                                  num_cores=2, num_subcores=16)
    return pl.kernel(
        body,
        out_type=jax.ShapeDtypeStruct((2, SN, ROWW), _f32),
        mesh=mesh,
        scratch_types=scratch,
        compiler_params=pltpu.CompilerParams(needs_layout_passes=False,
                                             use_tc_tiling_on_sc=False),
    )


def kernel(Corpus_, batch_inputs, entity_embeddings, relation_embed, edge_list,
           edge_type, edge_embed, edge_list_nhop, edge_type_nhop, a_head0,
           a2_head0, a_head1, a2_head1, a_out, a2_out, W):
    srcA = jnp.concatenate([edge_list[0], jnp.full((E1P - E1,), N_NODES, _i32)]).astype(_i32)
    dstA = jnp.concatenate([edge_list[1], jnp.zeros((E1P - E1,), _i32)]).astype(_i32)
    tyA = jnp.concatenate([edge_type, jnp.zeros((E1P - E1,), _i32)]).astype(_i32)
    srcB = jnp.concatenate([edge_list_nhop[0], jnp.full((E2P - E2,), N_NODES, _i32)]).astype(_i32)
    dstB = jnp.concatenate([edge_list_nhop[1], jnp.zeros((E2P - E2,), _i32)]).astype(_i32)
    t0B = jnp.concatenate([edge_type_nhop[:, 0], jnp.zeros((E2P - E2,), _i32)]).astype(_i32)
    t1B = jnp.concatenate([edge_type_nhop[:, 1], jnp.zeros((E2P - E2,), _i32)]).astype(_i32)
    idxa1 = jnp.stack([srcA.reshape(-1, C), dstA.reshape(-1, C)], axis=1)
    idxa2 = jnp.stack([srcA.reshape(-1, C), dstA.reshape(-1, C),
                       tyA.reshape(-1, C)], axis=1)
    idxb = jnp.stack([srcB.reshape(-1, C), dstB.reshape(-1, C),
                      t0B.reshape(-1, C), t1B.reshape(-1, C)], axis=1)

    usf, uss, udst, t3q, rfull, r2q, ortho = pl.pallas_call(
        _tc_a1,
        out_shape=[
            jax.ShapeDtypeStruct((SN, 64), _f32),
            jax.ShapeDtypeStruct((SN, 8), _f32),
            jax.ShapeDtypeStruct((SN, ROWW), _f32),
            jax.ShapeDtypeStruct((RP, ROWW), _f32),
            jax.ShapeDtypeStruct((NRELA, 64), _f32),
            jax.ShapeDtypeStruct((RP, ROWW), _f32),
            jax.ShapeDtypeStruct((1, 1), _f32),
        ],
    )(entity_embeddings, relation_embed, a_head0, a_head1, a2_head0, a2_head1,
      a_out, a2_out, W)

    BLK = 1280
    nblk = E1P // BLK
    nin = E1 // BLK
    gq1 = pl.pallas_call(
        _tc_a2,
        grid=(nblk,),
        in_specs=[
            pl.BlockSpec((BLK, RELDIM), lambda i: (jnp.minimum(i, nin - 1), 0)),
            pl.BlockSpec((1, NHID), lambda i: (0, 0)),
            pl.BlockSpec((1, NHID), lambda i: (0, 0)),
            pl.BlockSpec((NHID, 2 * NFEAT + RELDIM), lambda i: (0, 0)),
            pl.BlockSpec((NHID, 2 * NFEAT + RELDIM), lambda i: (0, 0)),
        ],
        out_specs=pl.BlockSpec((BLK, ROWW), lambda i: (i, 0)),
        out_shape=jax.ShapeDtypeStruct((E1P, ROWW), _f32),
    )(edge_embed, a2_head0, a2_head1, a_head0, a_head1)

    sc1 = _make_sc_kernel(2, LINEAR)
    acc1 = sc1(idxa1, gq1, idxb, t3q, uss, udst)

    vsf, vss, vdst = pl.pallas_call(
        _tc_b,
        out_shape=[
            jax.ShapeDtypeStruct((SN, 64), _f32),
            jax.ShapeDtypeStruct((SN, 8), _f32),
            jax.ShapeDtypeStruct((SN, ROWW), _f32),
        ],
    )(acc1, usf, a_out, a2_out)

    sc2 = _make_sc_kernel(1, G1)
    acc2 = sc2(idxa2, idxb, r2q, vss, vdst)

    x = pl.pallas_call(
        _tc_c,
        out_shape=jax.ShapeDtypeStruct((N_NODES, 64), _f32),
    )(acc2, vsf)

    return (x, rfull, ortho[0, 0])
